# baseline (speedup 1.0000x reference)
import functools

import jax
import jax.numpy as jnp
from jax import lax
from jax.experimental import pallas as pl
from jax.experimental.pallas import tpu as pltpu
from jax.experimental.pallas import tpu_sc as plsc

MAXREL = 128
D = 64
V = 2 * MAXREL + 1
L = 2048
AROWS = 2 * L

_NW = 32
_RPW = AROWS // _NW

NBUF = 8


def _sc_build_a_body(emb_hbm, a_hbm, idx_v, rows_v, sem):
    wid = lax.axis_index("s") * 2 + lax.axis_index("c")
    base = wid * _RPW
    for c in range(_RPW // 16):
        t = lax.iota(jnp.int32, 16) + (base + c * 16)
        idx_v[pl.ds(c * 16, 16)] = jnp.clip((L - 1 + MAXREL) - t, 0, V - 1)
    pltpu.async_copy(emb_hbm.at[idx_v], rows_v, sem).wait()
    pltpu.sync_copy(rows_v, a_hbm.at[pl.ds(base, _RPW)])


def _sc_build_a(embeddings):
    embp = jnp.pad(embeddings, ((0, 0), (0, 128 - D)))
    mesh = plsc.VectorSubcoreMesh(core_axis_name="c", subcore_axis_name="s")
    return pl.kernel(
        _sc_build_a_body,
        mesh=mesh,
        out_type=jax.ShapeDtypeStruct((AROWS, 128), jnp.float32),
        scratch_types=[
            pltpu.VMEM((_RPW,), jnp.int32),
            pltpu.VMEM((_RPW, 128), jnp.float32),
            pltpu.SemaphoreType.DMA,
        ],
    )(embp)


BR = 8


def _tc_window_body(a_ref, o_ref, a_t):
    @pl.when(pl.program_id(0) == 0)
    def _():
        for k in range(AROWS // 128):
            a_t[:, pl.ds(128 * k, 128)] = a_ref[pl.ds(128 * k, 128), 0:D].T

    i0 = pl.program_id(0) * BR
    for r in range(BR):
        s = (L - 1) - (i0 + r)
        q = s // 128
        rm = s % 128
        wfull = a_t[:, pl.ds(128 * q, L + 128)]
        o_ref[r] = pltpu.roll(wfull, (L + 128 - rm) % (L + 128), axis=1)[:, 0:L]


def _tc_fill(a):
    out = pl.pallas_call(
        _tc_window_body,
        grid=(L // BR,),
        in_specs=[pl.BlockSpec((AROWS, 128), lambda i: (0, 0))],
        out_specs=pl.BlockSpec((BR, D, L), lambda i: (i, 0, 0)),
        out_shape=jax.ShapeDtypeStruct((L, D, L), jnp.float32),
        scratch_shapes=[pltpu.VMEM((D, AROWS), jnp.float32)],
    )(a)
    return jnp.transpose(out, (0, 2, 1))


def kernel(inputs, embeddings):
    del inputs
    a = _sc_build_a(embeddings)
    return _tc_fill(a)


# speedup vs baseline: 1.0901x; 1.0901x over previous
import functools

import jax
import jax.numpy as jnp
from jax import lax
from jax.experimental import pallas as pl
from jax.experimental.pallas import tpu as pltpu
from jax.experimental.pallas import tpu_sc as plsc

MAXREL = 128
D = 64
V = 2 * MAXREL + 1
L = 2048
AROWS = 2 * L

_NW = 32
_RPW = AROWS // _NW

NBUF = 8


def _sc_build_a_body(emb_hbm, a_hbm, idx_v, rows_v, sem):
    wid = lax.axis_index("s") * 2 + lax.axis_index("c")
    base = wid * _RPW
    for c in range(_RPW // 16):
        t = lax.iota(jnp.int32, 16) + (base + c * 16)
        idx_v[pl.ds(c * 16, 16)] = jnp.clip((L - 1 + MAXREL) - t, 0, V - 1)
    pltpu.async_copy(emb_hbm.at[idx_v], rows_v, sem).wait()
    pltpu.sync_copy(rows_v, a_hbm.at[pl.ds(base, _RPW)])


def _sc_build_a(embeddings):
    embp = jnp.pad(embeddings, ((0, 0), (0, 128 - D)))
    mesh = plsc.VectorSubcoreMesh(core_axis_name="c", subcore_axis_name="s")
    return pl.kernel(
        _sc_build_a_body,
        mesh=mesh,
        out_type=jax.ShapeDtypeStruct((AROWS, 128), jnp.float32),
        scratch_types=[
            pltpu.VMEM((_RPW,), jnp.int32),
            pltpu.VMEM((_RPW, 128), jnp.float32),
            pltpu.SemaphoreType.DMA,
        ],
    )(embp)


BR = 16


def _tc_window_body(a_ref, o_ref, a_t):
    @pl.when(pl.program_id(0) == 0)
    def _():
        for k in range(AROWS // 128):
            a_t[:, pl.ds(128 * k, 128)] = a_ref[pl.ds(128 * k, 128), 0:D].T

    i0 = pl.program_id(0) * BR
    for r in range(BR):
        s = (L - 1) - (i0 + r)
        q = s // 128
        rm = s % 128
        wfull = a_t[:, pl.ds(128 * q, L + 128)]
        o_ref[r] = pltpu.roll(wfull, (L + 128 - rm) % (L + 128), axis=1)[:, 0:L]


def _tc_fill(a):
    out = pl.pallas_call(
        _tc_window_body,
        grid=(L // BR,),
        in_specs=[pl.BlockSpec((AROWS, 128), lambda i: (0, 0))],
        out_specs=pl.BlockSpec((BR, D, L), lambda i: (i, 0, 0)),
        out_shape=jax.ShapeDtypeStruct((L, D, L), jnp.float32),
        scratch_shapes=[pltpu.VMEM((D, AROWS), jnp.float32)],
    )(a)
    return jnp.transpose(out, (0, 2, 1))


def kernel(inputs, embeddings):
    del inputs
    a = _sc_build_a(embeddings)
    return _tc_fill(a)


# speedup vs baseline: 1.1207x; 1.0281x over previous
import functools

import jax
import jax.numpy as jnp
from jax import lax
from jax.experimental import pallas as pl
from jax.experimental.pallas import tpu as pltpu
from jax.experimental.pallas import tpu_sc as plsc

MAXREL = 128
D = 64
V = 2 * MAXREL + 1
L = 2048
AROWS = 2 * L

_NW = 32
_RPW = AROWS // _NW

NBUF = 8


def _sc_build_a_body(emb_hbm, a_hbm, idx_v, rows_v, sem):
    wid = lax.axis_index("s") * 2 + lax.axis_index("c")
    base = wid * _RPW
    for c in range(_RPW // 16):
        t = lax.iota(jnp.int32, 16) + (base + c * 16)
        idx_v[pl.ds(c * 16, 16)] = jnp.clip((L - 1 + MAXREL) - t, 0, V - 1)
    pltpu.async_copy(emb_hbm.at[idx_v], rows_v, sem).wait()
    pltpu.sync_copy(rows_v, a_hbm.at[pl.ds(base, _RPW)])


def _sc_build_a(embeddings):
    embp = jnp.pad(embeddings, ((0, 0), (0, 128 - D)))
    mesh = plsc.VectorSubcoreMesh(core_axis_name="c", subcore_axis_name="s")
    return pl.kernel(
        _sc_build_a_body,
        mesh=mesh,
        out_type=jax.ShapeDtypeStruct((AROWS, 128), jnp.float32),
        scratch_types=[
            pltpu.VMEM((_RPW,), jnp.int32),
            pltpu.VMEM((_RPW, 128), jnp.float32),
            pltpu.SemaphoreType.DMA,
        ],
    )(embp)


BR = 32


def _tc_window_body(a_ref, o_ref, a_t):
    @pl.when(pl.program_id(0) == 0)
    def _():
        for k in range(AROWS // 128):
            a_t[:, pl.ds(128 * k, 128)] = a_ref[pl.ds(128 * k, 128), 0:D].T

    i0 = pl.program_id(0) * BR
    for r in range(BR):
        s = (L - 1) - (i0 + r)
        q = s // 128
        rm = s % 128
        wfull = a_t[:, pl.ds(128 * q, L + 128)]
        o_ref[r] = pltpu.roll(wfull, (L + 128 - rm) % (L + 128), axis=1)[:, 0:L]


def _tc_fill(a):
    out = pl.pallas_call(
        _tc_window_body,
        grid=(L // BR,),
        in_specs=[pl.BlockSpec((AROWS, 128), lambda i: (0, 0))],
        out_specs=pl.BlockSpec((BR, D, L), lambda i: (i, 0, 0)),
        out_shape=jax.ShapeDtypeStruct((L, D, L), jnp.float32),
        scratch_shapes=[pltpu.VMEM((D, AROWS), jnp.float32)],
    )(a)
    return jnp.transpose(out, (0, 2, 1))


def kernel(inputs, embeddings):
    del inputs
    a = _sc_build_a(embeddings)
    return _tc_fill(a)


# speedup vs baseline: 1.2745x; 1.1372x over previous
import functools

import jax
import jax.numpy as jnp
from jax import lax
from jax.experimental import pallas as pl
from jax.experimental.pallas import tpu as pltpu
from jax.experimental.pallas import tpu_sc as plsc

MAXREL = 128
D = 64
V = 2 * MAXREL + 1
L = 2048
AROWS = 2 * L

_NW = 32
_RPW = AROWS // _NW

NBUF = 8


def _sc_build_a_body(emb_hbm, a_hbm, idx_v, rows_v, sem):
    wid = lax.axis_index("s") * 2 + lax.axis_index("c")
    base = wid * _RPW
    for c in range(_RPW // 16):
        t = lax.iota(jnp.int32, 16) + (base + c * 16)
        idx_v[pl.ds(c * 16, 16)] = jnp.clip((L - 1 + MAXREL) - t, 0, V - 1)
    pltpu.async_copy(emb_hbm.at[idx_v], rows_v, sem).wait()
    pltpu.sync_copy(rows_v, a_hbm.at[pl.ds(base, _RPW)])


def _sc_build_a(embeddings):
    embp = jnp.pad(embeddings, ((0, 0), (0, 128 - D)))
    mesh = plsc.VectorSubcoreMesh(core_axis_name="c", subcore_axis_name="s")
    return pl.kernel(
        _sc_build_a_body,
        mesh=mesh,
        out_type=jax.ShapeDtypeStruct((AROWS, 128), jnp.float32),
        scratch_types=[
            pltpu.VMEM((_RPW,), jnp.int32),
            pltpu.VMEM((_RPW, 128), jnp.float32),
            pltpu.SemaphoreType.DMA,
        ],
    )(embp)


BR = 32


def _tc_window_body(a_ref, o_ref, a_t):
    @pl.when(pl.program_id(0) == 0)
    def _():
        for k in range(AROWS // 128):
            a_t[:, pl.ds(128 * k, 128)] = a_ref[pl.ds(128 * k, 128), 0:D].T

    i0 = pl.program_id(0) * BR
    for r in range(BR):
        s = (L - 1) - (i0 + r)
        q = s // 128
        rm = s % 128
        o_ref[r] = a_t[:, pl.ds(128 * q, L)]
        tb = jnp.clip((1919 - s) // 128, 0, 10)
        wband = a_t[:, pl.ds(128 * (q + tb), 896)]
        band = pltpu.roll(wband, (896 - rm) % 896, axis=1)[:, 0:768]
        o_ref[r, :, pl.ds(128 * tb, 768)] = band


def _tc_fill(a):
    out = pl.pallas_call(
        _tc_window_body,
        grid=(L // BR,),
        in_specs=[pl.BlockSpec((AROWS, 128), lambda i: (0, 0))],
        out_specs=pl.BlockSpec((BR, D, L), lambda i: (i, 0, 0)),
        out_shape=jax.ShapeDtypeStruct((L, D, L), jnp.float32),
        scratch_shapes=[pltpu.VMEM((D, AROWS), jnp.float32)],
    )(a)
    return jnp.transpose(out, (0, 2, 1))


def kernel(inputs, embeddings):
    del inputs
    a = _sc_build_a(embeddings)
    return _tc_fill(a)


# speedup vs baseline: 1.2751x; 1.0005x over previous
import jax
import jax.numpy as jnp
from jax import lax
from jax.experimental import pallas as pl
from jax.experimental.pallas import tpu as pltpu
from jax.experimental.pallas import tpu_sc as plsc

MAXREL = 128
D = 64
V = 2 * MAXREL + 1
L = 2048
AROWS = 2 * L

_NW = 32
_RPW = AROWS // _NW


def _sc_build_a_body(emb_hbm, a_hbm, idx_v, rows_v, sem):
    wid = lax.axis_index("s") * 2 + lax.axis_index("c")
    base = wid * _RPW
    for c in range(_RPW // 16):
        t = lax.iota(jnp.int32, 16) + (base + c * 16)
        idx_v[pl.ds(c * 16, 16)] = jnp.clip((L - 1 + MAXREL) - t, 0, V - 1)
    pltpu.async_copy(emb_hbm.at[idx_v], rows_v, sem).wait()
    pltpu.sync_copy(rows_v, a_hbm.at[pl.ds(base, _RPW)])


def _sc_build_a(embeddings):
    embp = jnp.pad(embeddings, ((0, 0), (0, 128 - D)))
    mesh = plsc.VectorSubcoreMesh(core_axis_name="c", subcore_axis_name="s")
    return pl.kernel(
        _sc_build_a_body,
        mesh=mesh,
        out_type=jax.ShapeDtypeStruct((AROWS, 128), jnp.float32),
        scratch_types=[
            pltpu.VMEM((_RPW,), jnp.int32),
            pltpu.VMEM((_RPW, 128), jnp.float32),
            pltpu.SemaphoreType.DMA,
        ],
    )(embp)


BR = 32


def _tc_window_body(a_ref, o_ref, a_t):
    @pl.when(pl.program_id(0) == 0)
    def _():
        for k in range(AROWS // 128):
            a_t[:, pl.ds(128 * k, 128)] = a_ref[pl.ds(128 * k, 128), 0:D].T

    i0 = pl.program_id(0) * BR
    for r in range(BR):
        s = (L - 1) - (i0 + r)
        q = s // 128
        rm = s % 128
        o_ref[r] = a_t[:, pl.ds(128 * q, L)]
        tb = jnp.clip((1919 - s) // 128, 0, 10)
        wband = a_t[:, pl.ds(128 * (q + tb), 896)]
        band = pltpu.roll(wband, (896 - rm) % 896, axis=1)[:, 0:768]
        o_ref[r, :, pl.ds(128 * tb, 768)] = band


def _tc_fill(a):
    out = pl.pallas_call(
        _tc_window_body,
        grid=(L // BR,),
        in_specs=[pl.BlockSpec((AROWS, 128), lambda i: (0, 0))],
        out_specs=pl.BlockSpec((BR, D, L), lambda i: (i, 0, 0)),
        out_shape=jax.ShapeDtypeStruct((L, D, L), jnp.float32),
        scratch_shapes=[pltpu.VMEM((D, AROWS), jnp.float32)],
    )(a)
    return jnp.transpose(out, (0, 2, 1))


def kernel(inputs, embeddings):
    del inputs
    a = _sc_build_a(embeddings)
    return _tc_fill(a)
